# Initial kernel scaffold; baseline (speedup 1.0000x reference)
import jax
import jax.numpy as jnp
from jax.experimental import pallas as pl


def kernel(X, NX, EW, node_emb, edge_w, node_w, fc_W, fc_b):
    raise NotImplementedError("write your pallas kernel here")


# speedup vs baseline: 21.9703x; 21.9703x over previous
import functools

import jax
import jax.numpy as jnp
import numpy as np
from jax import lax
from jax.experimental import pallas as pl
from jax.experimental.pallas import tpu as pltpu
from jax.experimental.pallas import tpu_sc as plsc

NUM_NODES = 10000
QROWS = NUM_NODES + 1
D = 128
C = 20
B = 1024
L = 50
W = 10

QW = 32
NEI = L * W
NEI_PAD = 512
SELF_PAD = 64
NTERMS = NEI_PAD + SELF_PAD
NC = 2
NS = 16
NW_WORKERS = NC * NS
B_PER_W = B // NW_WORKERS

_LANES = 16


def _build_q_tc(node_emb, fc_w_pad):
    blk = 1000

    def body(emb_ref, fcw_ref, out_ref):
        out_ref[...] = lax.dot_general(
            emb_ref[...], fcw_ref[...],
            dimension_numbers=(((1,), (1,)), ((), ())),
            preferred_element_type=jnp.float32)

    return pl.pallas_call(
        body,
        grid=(NUM_NODES // blk,),
        in_specs=[
            pl.BlockSpec((blk, D), lambda i: (i, 0)),
            pl.BlockSpec((QW, D), lambda i: (0, 0)),
        ],
        out_specs=pl.BlockSpec((blk, QW), lambda i: (i, 0)),
        out_shape=jax.ShapeDtypeStruct((NUM_NODES, QW), jnp.float32),
    )(node_emb, fc_w_pad)


def _sc_body(q_hbm, idxn_hbm, idxs_hbm, lmap_hbm, out_hbm,
             idx4_v, idxs_v, lmap_v, nw_v, r_v, t_v, out_v, sem):
    wid = lax.axis_index("s") * NC + lax.axis_index("c")
    iota = lax.iota(jnp.int32, _LANES)
    zeros16 = jnp.zeros((_LANES,), jnp.float32)

    pltpu.sync_copy(lmap_hbm, lmap_v)
    for r in range(QW):
        t_v[r, :] = zeros16

    def per_row(i, _):
        b = wid * B_PER_W + i
        pltpu.sync_copy(idxn_hbm.at[b], idx4_v)
        pltpu.sync_copy(idxs_hbm.at[b], idxs_v)
        cps = [pltpu.async_copy(q_hbm.at[idx4_v.at[j]],
                                r_v.at[pl.ds(j * 128, 128)], sem)
               for j in range(4)]
        cps.append(pltpu.async_copy(q_hbm.at[idxs_v],
                                    r_v.at[pl.ds(NEI_PAD, SELF_PAD)], sem))
        for cp in cps:
            cp.wait()

        for k2 in range(SELF_PAD // _LANES):
            rows = (NEI_PAD + k2 * _LANES) + iota
            nw_v[pl.ds(k2 * _LANES, _LANES)] = plsc.load_gather(
                r_v, [rows, jnp.full((_LANES,), 21, jnp.int32)])

        acc0 = tuple(zeros16 for _ in range(C))

        def nei_block(k, acc):
            tvec = k * _LANES + iota
            lvec = lmap_v[pl.ds(k * _LANES, _LANES)]
            nwg = plsc.load_gather(nw_v, [lvec])
            ew = plsc.load_gather(
                r_v, [tvec, jnp.full((_LANES,), 20, jnp.int32)])
            c = (1.0 - nwg) * ew
            return tuple(
                acc[d] + c * plsc.load_gather(
                    r_v, [tvec, jnp.full((_LANES,), d, jnp.int32)])
                for d in range(C))

        acc = lax.fori_loop(0, NEI_PAD // _LANES, nei_block, acc0)

        for k2 in range(SELF_PAD // _LANES):
            tvec = (NEI_PAD + k2 * _LANES) + iota
            c = nw_v[pl.ds(k2 * _LANES, _LANES)]
            acc = tuple(
                acc[d] + c * plsc.load_gather(
                    r_v, [tvec, jnp.full((_LANES,), d, jnp.int32)])
                for d in range(C))

        for d in range(C):
            t_v[d, :] = acc[d]
        hv0 = zeros16
        hv1 = zeros16
        for j in range(_LANES):
            jf = jnp.full((_LANES,), j, jnp.int32)
            hv0 = hv0 + plsc.load_gather(t_v, [iota, jf])
            hv1 = hv1 + plsc.load_gather(t_v, [_LANES + iota, jf])

        h0 = jnp.maximum(hv0, 0.0)
        h1 = jnp.maximum(hv1, 0.0)
        valid1 = iota < (C - _LANES)
        h1m = jnp.where(valid1, h1, -30.0)
        m = jnp.maximum(jnp.max(h0), jnp.max(h1m))
        e0 = jnp.exp(h0 - m)
        e1 = jnp.where(valid1, jnp.exp(h1 - m), 0.0)
        s = jnp.sum(e0) + jnp.sum(e1)
        out_v[i, pl.ds(0, _LANES)] = e0 / s
        out_v[i, pl.ds(_LANES, _LANES)] = e1 / s
        return 0

    lax.fori_loop(0, B_PER_W, per_row, 0)
    pltpu.sync_copy(out_v, out_hbm.at[pl.ds(wid * B_PER_W, B_PER_W)])


@functools.lru_cache(maxsize=1)
def _sc_kernel():
    return pl.kernel(
        _sc_body,
        out_type=jax.ShapeDtypeStruct((B, QW), jnp.float32),
        mesh=plsc.VectorSubcoreMesh(core_axis_name="c", subcore_axis_name="s"),
        compiler_params=pltpu.CompilerParams(needs_layout_passes=False,
                                             use_tc_tiling_on_sc=False),
        scratch_types=[
            pltpu.VMEM((4, 128), jnp.int32),
            pltpu.VMEM((SELF_PAD,), jnp.int32),
            pltpu.VMEM((NEI_PAD,), jnp.int32),
            pltpu.VMEM((SELF_PAD,), jnp.float32),
            pltpu.VMEM((NTERMS, QW), jnp.float32),
            pltpu.VMEM((QW, _LANES), jnp.float32),
            pltpu.VMEM((B_PER_W, QW), jnp.float32),
            pltpu.SemaphoreType.DMA,
        ],
    )

_LMAP_NP = np.where(
    np.arange(NEI_PAD) < NEI, np.arange(NEI_PAD) // W, 0).astype(np.int32)


def kernel(X, NX, EW, node_emb, edge_w, node_w, fc_W, fc_b):
    del EW
    x32 = X.astype(jnp.int32)
    nx32 = NX.astype(jnp.int32)

    fc_w_pad = jnp.pad(fc_W, ((0, QW - C), (0, 0)))
    p = _build_q_tc(node_emb, fc_w_pad)
    ew10k = lax.slice(edge_w, (0, 0), (NUM_NODES, 1))
    q_main = jnp.concatenate(
        [p[:, :C], ew10k, node_w, jnp.zeros((NUM_NODES, QW - C - 2),
                                            jnp.float32)], axis=1)
    bias_row = jnp.concatenate(
        [fc_b, jnp.zeros((1,), jnp.float32), jnp.ones((1,), jnp.float32),
         jnp.zeros((QW - C - 2,), jnp.float32)]).reshape(1, QW)
    q = jnp.concatenate([q_main, bias_row], axis=0)

    idxn = jnp.concatenate(
        [nx32.reshape(B, NEI),
         jnp.zeros((B, NEI_PAD - NEI), jnp.int32)], axis=1).reshape(B, 4, 128)
    idxs = jnp.concatenate(
        [x32, jnp.full((B, 1), NUM_NODES, jnp.int32),
         jnp.zeros((B, SELF_PAD - L - 1), jnp.int32)], axis=1)

    out = _sc_kernel()(q, idxn, idxs, jnp.asarray(_LMAP_NP))
    return out[:, :C]
